# baseline (speedup 1.0000x reference)
import functools

import jax
import jax.numpy as jnp
from jax import lax
from jax.experimental import pallas as pl
from jax.experimental.pallas import tpu as pltpu
from jax.experimental.pallas import tpu_sc as plsc

D = 32
B = 16384
K = 5
NC = 2
NS = 16
L = 16
NW = NC * NS
BW = B // NW
CH = 128
NCH = BW // CH
NEG_BW = BW * K
NEG_NCH = NEG_BW // CH
GROUPS = BW // L


def _sc_body(vi_idx_hbm, vj_idx_hbm, neg_idx_hbm, second_hbm, context_hbm,
             dots_hbm,
             vi_idx, vj_idx, neg_idx, vi_rows, vj_rows, neg_rows, dots_st,
             sem):
  wid = lax.axis_index("s") * NC + lax.axis_index("c")

  pltpu.sync_copy(vi_idx_hbm.at[wid], vi_idx)
  pltpu.sync_copy(vj_idx_hbm.at[wid], vj_idx)
  pltpu.sync_copy(neg_idx_hbm.at[wid], neg_idx)

  copies = []
  for c in range(NCH):
    copies.append(pltpu.async_copy(
        second_hbm.at[vi_idx.at[c]], vi_rows.at[pl.ds(c * CH, CH)], sem))
    copies.append(pltpu.async_copy(
        context_hbm.at[vj_idx.at[c]], vj_rows.at[pl.ds(c * CH, CH)], sem))
  for c in range(NEG_NCH):
    copies.append(pltpu.async_copy(
        context_hbm.at[neg_idx.at[c]], neg_rows.at[pl.ds(c * CH, CH)], sem))
  for cp in copies:
    cp.wait()

  iota = lax.iota(jnp.int32, L)
  zeros = jnp.zeros((L,), jnp.float32)

  def group(g, carry):
    rows = g * L + iota
    nrows = rows * K
    cols = [jnp.full((L,), d, jnp.int32) for d in range(D)]
    vic = [plsc.load_gather(vi_rows, [rows, cols[d]]) for d in range(D)]

    acc = zeros
    for d in range(D):
      acc = acc + vic[d] * plsc.load_gather(vj_rows, [rows, cols[d]])
    dots_st[0, pl.ds(g * L, L)] = acc

    for k in range(K):
      nr = nrows + k
      acc = zeros
      for d in range(D):
        acc = acc - vic[d] * plsc.load_gather(neg_rows, [nr, cols[d]])
      dots_st[1 + k, pl.ds(g * L, L)] = acc
    return carry

  lax.fori_loop(0, GROUPS, group, 0)

  pltpu.sync_copy(dots_st, dots_hbm.at[wid])


@functools.partial(
    pl.kernel,
    out_type=jax.ShapeDtypeStruct((NW, 1 + K, BW), jnp.float32),
    mesh=plsc.VectorSubcoreMesh(core_axis_name="c", subcore_axis_name="s",
                                num_cores=NC, num_subcores=NS),
    compiler_params=pltpu.CompilerParams(needs_layout_passes=False,
                                         use_tc_tiling_on_sc=False),
    scratch_types=[
        pltpu.VMEM((NCH, CH), jnp.int32),
        pltpu.VMEM((NCH, CH), jnp.int32),
        pltpu.VMEM((NEG_NCH, CH), jnp.int32),
        pltpu.VMEM((BW, D), jnp.float32),
        pltpu.VMEM((BW, D), jnp.float32),
        pltpu.VMEM((NEG_BW, D), jnp.float32),
        pltpu.VMEM((1 + K, BW), jnp.float32),
        pltpu.SemaphoreType.DMA,
    ],
)
def _sc_dots(*args):
  _sc_body(*args)


def _tc_body(x_ref, o_ref):
  x = x_ref[...]
  y = jnp.minimum(x, 0.0) - jnp.log1p(jnp.exp(-jnp.abs(x)))
  o_ref[...] = jnp.full((1, 1), -1.0 / B) * jnp.sum(y)


_tc_loss = pl.pallas_call(
    _tc_body,
    out_shape=jax.ShapeDtypeStruct((1, 1), jnp.float32),
)


def kernel(v_i, v_j, negsamples, second_embeddings, context_embeddings):
  vi_idx = v_i.astype(jnp.int32).reshape(NW, NCH, CH)
  vj_idx = v_j.astype(jnp.int32).reshape(NW, NCH, CH)
  neg_idx = negsamples.astype(jnp.int32).reshape(NW, NEG_NCH, CH)
  dots = _sc_dots(vi_idx, vj_idx, neg_idx, second_embeddings,
                  context_embeddings)
  loss = _tc_loss(dots.reshape(NW * (1 + K), BW))
  return loss[0, 0]


# speedup vs baseline: 1.3998x; 1.3998x over previous
import functools

import jax
import jax.numpy as jnp
from jax import lax
from jax.experimental import pallas as pl
from jax.experimental.pallas import tpu as pltpu
from jax.experimental.pallas import tpu_sc as plsc

D = 32
B = 16384
K = 5
NC = 2
NS = 16
L = 16
NW = NC * NS
BW = B // NW
G = BW // L
NEG_G = K * L


def _sc_body(vi_idx_hbm, vj_idx_hbm, neg_idx_hbm, second_hbm, context_hbm,
             dots_hbm,
             vi_idx, vj_idx, neg_idx,
             bvi0, bvj0, bneg0, bvi1, bvj1, bneg1,
             dots_st, sem_a, sem_b):
  wid = lax.axis_index("s") * NC + lax.axis_index("c")

  pltpu.sync_copy(vi_idx_hbm.at[wid], vi_idx)
  pltpu.sync_copy(vj_idx_hbm.at[wid], vj_idx)
  pltpu.sync_copy(neg_idx_hbm.at[wid], neg_idx)

  iota = lax.iota(jnp.int32, L)
  cols = [jnp.full((L,), d, jnp.int32) for d in range(D)]
  nrows = [iota * K + k for k in range(K)]
  zeros = jnp.zeros((L,), jnp.float32)

  def fire(g, bvi, bvj, bneg, sem):
    iv = vi_idx[g, :]
    jv = vj_idx[g, :]
    for j in range(L):
      pltpu.async_copy(second_hbm.at[pl.ds(iv[j], 1)],
                       bvi.at[pl.ds(j, 1)], sem)
      pltpu.async_copy(context_hbm.at[pl.ds(jv[j], 1)],
                       bvj.at[pl.ds(j, 1)], sem)
    for c in range(K):
      nv = neg_idx[g, pl.ds(c * L, L)]
      for j in range(L):
        pltpu.async_copy(context_hbm.at[pl.ds(nv[j], 1)],
                         bneg.at[pl.ds(c * L + j, 1)], sem)

  def drain(bvi, bvj, bneg, sem):
    pltpu.make_async_copy(second_hbm.at[pl.ds(0, L)], bvi, sem).wait()
    pltpu.make_async_copy(context_hbm.at[pl.ds(0, L)], bvj, sem).wait()
    pltpu.make_async_copy(context_hbm.at[pl.ds(0, NEG_G)], bneg, sem).wait()

  def compute(g, bvi, bvj, bneg):
    vic = [plsc.load_gather(bvi, [iota, cols[d]]) for d in range(D)]
    acc = zeros
    for d in range(D):
      acc = acc + vic[d] * plsc.load_gather(bvj, [iota, cols[d]])
    dots_st[0, pl.ds(g * L, L)] = acc
    for k in range(K):
      acc = zeros
      for d in range(D):
        acc = acc - vic[d] * plsc.load_gather(bneg, [nrows[k], cols[d]])
      dots_st[1 + k, pl.ds(g * L, L)] = acc

  def body(g, carry):
    even = g % 2 == 0

    @pl.when(jnp.logical_and(g < G, even))
    def _():
      fire(g, bvi0, bvj0, bneg0, sem_a)

    @pl.when(jnp.logical_and(g < G, jnp.logical_not(even)))
    def _():
      fire(g, bvi1, bvj1, bneg1, sem_b)

    @pl.when(jnp.logical_and(g > 0, even))
    def _():
      drain(bvi1, bvj1, bneg1, sem_b)
      compute(g - 1, bvi1, bvj1, bneg1)

    @pl.when(jnp.logical_and(g > 0, jnp.logical_not(even)))
    def _():
      drain(bvi0, bvj0, bneg0, sem_a)
      compute(g - 1, bvi0, bvj0, bneg0)

    return carry

  lax.fori_loop(0, G + 1, body, 0)

  pltpu.sync_copy(dots_st, dots_hbm.at[wid])


@functools.partial(
    pl.kernel,
    out_type=jax.ShapeDtypeStruct((NW, 1 + K, BW), jnp.float32),
    mesh=plsc.VectorSubcoreMesh(core_axis_name="c", subcore_axis_name="s",
                                num_cores=NC, num_subcores=NS),
    compiler_params=pltpu.CompilerParams(needs_layout_passes=False),
    scratch_types=[
        pltpu.VMEM((G, L), jnp.int32),
        pltpu.VMEM((G, L), jnp.int32),
        pltpu.VMEM((G, NEG_G), jnp.int32),
        pltpu.VMEM((L, D), jnp.float32),
        pltpu.VMEM((L, D), jnp.float32),
        pltpu.VMEM((NEG_G, D), jnp.float32),
        pltpu.VMEM((L, D), jnp.float32),
        pltpu.VMEM((L, D), jnp.float32),
        pltpu.VMEM((NEG_G, D), jnp.float32),
        pltpu.VMEM((1 + K, BW), jnp.float32),
        pltpu.SemaphoreType.DMA,
        pltpu.SemaphoreType.DMA,
    ],
)
def _sc_dots(*args):
  _sc_body(*args)


def _tc_body(x_ref, o_ref):
  x = x_ref[...]
  y = jnp.minimum(x, 0.0) - jnp.log1p(jnp.exp(-jnp.abs(x)))
  o_ref[...] = jnp.full((1, 1), -1.0 / B) * jnp.sum(y)


_tc_loss = pl.pallas_call(
    _tc_body,
    out_shape=jax.ShapeDtypeStruct((1, 1), jnp.float32),
)


def kernel(v_i, v_j, negsamples, second_embeddings, context_embeddings):
  vi_idx = v_i.astype(jnp.int32).reshape(NW, G, L)
  vj_idx = v_j.astype(jnp.int32).reshape(NW, G, L)
  neg_idx = negsamples.astype(jnp.int32).reshape(NW, G, NEG_G)
  dots = _sc_dots(vi_idx, vj_idx, neg_idx, second_embeddings,
                  context_embeddings)
  loss = _tc_loss(dots.reshape(NW * (1 + K), BW))
  return loss[0, 0]
